# baseline (speedup 1.0000x reference)
import functools

import jax
import jax.numpy as jnp
from jax import lax
from jax.experimental import pallas as pl
from jax.experimental.pallas import tpu as pltpu
from jax.experimental.pallas import tpu_sc as plsc

_LAMBDA_PARAM = 0.001
_LAMBDA_ADV = 1.0
_EPSILON = 0.5
_EPOCH_ADV = 1

_NC = 2
_NS = 16
_NW = _NC * _NS
_SLAB = 16


def _make_sc_gather(B, D):
    b_per_w = B // _NW
    out_t = jax.ShapeDtypeStruct((D, B), jnp.float32)
    mesh = plsc.VectorSubcoreMesh(core_axis_name="c", subcore_axis_name="s")

    @functools.partial(
        pl.kernel,
        mesh=mesh,
        out_type=[out_t, out_t, out_t],
        compiler_params=pltpu.CompilerParams(needs_layout_passes=False),
        scratch_types=[
            pltpu.VMEM((b_per_w,), jnp.int32),
            pltpu.VMEM((b_per_w,), jnp.int32),
            pltpu.VMEM((b_per_w,), jnp.int32),
            pltpu.VMEM((16, 3 * _SLAB * 128), jnp.float32),
            pltpu.VMEM((D, b_per_w), jnp.float32),
            pltpu.VMEM((D, b_per_w), jnp.float32),
            pltpu.VMEM((D, b_per_w), jnp.float32),
            pltpu.SemaphoreType.DMA,
            pltpu.SemaphoreType.DMA,
            pltpu.SemaphoreType.DMA,
            pltpu.SemaphoreType.DMA,
            pltpu.SemaphoreType.DMA,
            pltpu.SemaphoreType.DMA,
        ],
    )
    def gather_k(user_hbm, itemi_hbm, itemj_hbm, eu_hbm, ei_hbm,
                 u_out, i_out, j_out,
                 idx_u, idx_i, idx_j, stage, fm_u, fm_i, fm_j,
                 sem0, sem1, sem2, sem3, sem4, sem5):
        wid = lax.axis_index("s") * _NC + lax.axis_index("c")
        base = wid * b_per_w
        in_sl = pl.ds(base, b_per_w)
        pltpu.sync_copy(user_hbm.at[in_sl], idx_u)
        pltpu.sync_copy(itemi_hbm.at[in_sl], idx_i)
        pltpu.sync_copy(itemj_hbm.at[in_sl], idx_j)
        iota = lax.iota(jnp.int32, 16)
        n_slabs = b_per_w // _SLAB
        sems = ((sem0, sem1), (sem2, sem3), (sem4, sem5))
        half_bytes = _SLAB // 2

        def fire(idx_v, tbl, t, third):
            rv = idx_v[pl.ds(t * _SLAB, _SLAB)]
            r128v = (rv >> 7) << 7
            for s in range(_SLAB):
                r128 = pl.multiple_of(r128v[s], 128)
                pltpu.async_copy(
                    tbl.at[:, pl.ds(r128, 128)],
                    stage.at[:, pl.ds((third * _SLAB + s) * 128, 128)],
                    sems[third][s % 2])

        def drain_extract(idx_v, tbl, fm, t, third):
            rv = idx_v[pl.ds(t * _SLAB, _SLAB)]
            for q in range(2):
                pltpu.make_async_copy(
                    tbl.at[:, pl.ds(0, half_bytes * 128)],
                    stage.at[:, pl.ds(third * _SLAB * 128, half_bytes * 128)],
                    sems[third][q]).wait()
            lane = (rv & 127) + (third * _SLAB * 128)
            for s in range(_SLAB):
                col = plsc.load_gather(
                    stage,
                    [iota, jnp.full((16,), lane[s] + s * 128, jnp.int32)])
                plsc.store_scatter(
                    fm, [iota, jnp.full((16,), t * _SLAB + s, jnp.int32)], col)

        work = ((idx_u, eu_hbm, fm_u), (idx_i, ei_hbm, fm_i),
                (idx_j, ei_hbm, fm_j))

        for idx_v, tbl, fm in work:
            fire(idx_v, tbl, 0, 0)
            fire(idx_v, tbl, 1, 1)

            def body(p, _, idx_v=idx_v, tbl=tbl, fm=fm):
                t0 = p * 3
                for k in range(3):
                    t = t0 + k
                    nxt = t + 2

                    @pl.when(nxt < n_slabs)
                    def _(t=t, nxt=nxt, k=k):
                        fire(idx_v, tbl, nxt, (k + 2) % 3)

                    @pl.when(t < n_slabs)
                    def _(t=t, k=k):
                        drain_extract(idx_v, tbl, fm, t, k)

                return 0

            lax.fori_loop(0, (n_slabs + 2) // 3, body, 0)

        out_sl = pl.ds(base, b_per_w)
        pltpu.sync_copy(fm_u, u_out.at[:, out_sl])
        pltpu.sync_copy(fm_i, i_out.at[:, out_sl])
        pltpu.sync_copy(fm_j, j_out.at[:, out_sl])

    return gather_k


def _loss_body(u_ref, i_ref, j_ref, out_ref, acc):
    b = pl.program_id(0)
    nb = pl.num_programs(0)

    @pl.when(b == 0)
    def _init():
        acc[0] = 0.0
        acc[1] = 0.0

    u = u_ref[...]
    i = i_ref[...]
    j = j_ref[...]

    x_ui = jnp.sum(u * i, axis=0, keepdims=True)
    x_uj = jnp.sum(u * j, axis=0, keepdims=True)
    t = x_ui - x_uj
    x = jnp.clip(t, -80.0, 1e8)
    log_prob = jnp.sum(jax.nn.log_sigmoid(x))
    reg = _LAMBDA_PARAM * (jnp.sum(u * u) + jnp.sum(i * i) + jnp.sum(j * j))

    mask = ((t >= -80.0) & (t <= 1e8)).astype(jnp.float32)
    c = jax.nn.sigmoid(-x) * mask
    gu = -c * (i - j) + (2.0 * _LAMBDA_PARAM) * u
    gi = -c * u + (2.0 * _LAMBDA_PARAM) * i
    gj = c * u + (2.0 * _LAMBDA_PARAM) * j

    def _delta(g):
        n = jnp.maximum(jnp.sqrt(jnp.sum(g * g, axis=0, keepdims=True)), 1e-12)
        return (_EPSILON / n) * g

    ua = u + _delta(gu)
    x_ui_a = jnp.sum(ua * (i + _delta(gi)), axis=0, keepdims=True)
    x_uj_a = jnp.sum(ua * (j + _delta(gj)), axis=0, keepdims=True)
    x_a = jnp.clip(x_ui_a - x_uj_a, -80.0, 1e8)
    log_prob_adv = jnp.sum(jax.nn.log_sigmoid(x_a))

    acc[0] += -log_prob + reg
    acc[1] += -_LAMBDA_ADV * log_prob_adv

    @pl.when(b == nb - 1)
    def _emit():
        loss = acc[0]
        out_ref[0] = loss
        out_ref[1] = loss + acc[1]


def _make_tc_loss(B, D, block):
    grid = B // block
    spec = pl.BlockSpec((D, block), lambda b: (0, b))
    return pl.pallas_call(
        _loss_body,
        grid=(grid,),
        in_specs=[spec, spec, spec],
        out_specs=pl.BlockSpec(memory_space=pltpu.SMEM),
        out_shape=jax.ShapeDtypeStruct((2,), jnp.float32),
        scratch_shapes=[pltpu.SMEM((2,), jnp.float32)],
    )


def kernel(user, item_i, item_j, epoch, embed_user, embed_item):
    B, = user.shape
    D = embed_user.shape[1]
    u_fm, i_fm, j_fm = _make_sc_gather(B, D)(
        user.astype(jnp.int32), item_i.astype(jnp.int32),
        item_j.astype(jnp.int32), embed_user.T, embed_item.T)
    out = _make_tc_loss(B, D, B)(u_fm, i_fm, j_fm)
    return jnp.where(epoch < _EPOCH_ADV, out[0], out[1])


# speedup vs baseline: 1.0309x; 1.0309x over previous
import functools

import jax
import jax.numpy as jnp
from jax import lax
from jax.experimental import pallas as pl
from jax.experimental.pallas import tpu as pltpu
from jax.experimental.pallas import tpu_sc as plsc

_LAMBDA_PARAM = 0.001
_LAMBDA_ADV = 1.0
_EPSILON = 0.5
_EPOCH_ADV = 1

_NC = 2
_NS = 16
_NW = _NC * _NS
_SLAB = 16


def _make_sc_gather(B, D):
    b_per_w = B // _NW
    out_t = jax.ShapeDtypeStruct((D, B), jnp.float32)
    mesh = plsc.VectorSubcoreMesh(core_axis_name="c", subcore_axis_name="s")

    @functools.partial(
        pl.kernel,
        mesh=mesh,
        out_type=[out_t, out_t, out_t],
        compiler_params=pltpu.CompilerParams(needs_layout_passes=False),
        scratch_types=[
            pltpu.VMEM((b_per_w,), jnp.int32),
            pltpu.VMEM((b_per_w,), jnp.int32),
            pltpu.VMEM((b_per_w,), jnp.int32),
            pltpu.VMEM((16, 3 * _SLAB * 128), jnp.float32),
            pltpu.VMEM((D, b_per_w), jnp.float32),
            pltpu.VMEM((D, b_per_w), jnp.float32),
            pltpu.VMEM((D, b_per_w), jnp.float32),
            pltpu.SemaphoreType.DMA,
            pltpu.SemaphoreType.DMA,
            pltpu.SemaphoreType.DMA,
            pltpu.SemaphoreType.DMA,
            pltpu.SemaphoreType.DMA,
            pltpu.SemaphoreType.DMA,
        ],
    )
    def gather_k(user_hbm, itemi_hbm, itemj_hbm, eu_hbm, ei_hbm,
                 u_out, i_out, j_out,
                 idx_u, idx_i, idx_j, stage, fm_u, fm_i, fm_j,
                 sem0, sem1, sem2, sem3, sem4, sem5):
        wid = lax.axis_index("s") * _NC + lax.axis_index("c")
        base = wid * b_per_w
        in_sl = pl.ds(base, b_per_w)
        pltpu.sync_copy(user_hbm.at[in_sl], idx_u)
        pltpu.sync_copy(itemi_hbm.at[in_sl], idx_i)
        pltpu.sync_copy(itemj_hbm.at[in_sl], idx_j)
        iota = lax.iota(jnp.int32, 16)
        n_slabs = b_per_w // _SLAB
        sems = ((sem0, sem1), (sem2, sem3), (sem4, sem5))
        half_bytes = _SLAB // 2

        def fire(idx_v, tbl, t, third):
            rv = idx_v[pl.ds(t * _SLAB, _SLAB)]
            r128v = (rv >> 7) << 7
            for s in range(_SLAB):
                r128 = pl.multiple_of(r128v[s], 128)
                pltpu.async_copy(
                    tbl.at[:, pl.ds(r128, 128)],
                    stage.at[:, pl.ds((third * _SLAB + s) * 128, 128)],
                    sems[third][s % 2])

        def drain_extract(idx_v, tbl, fm, t, third):
            rv = idx_v[pl.ds(t * _SLAB, _SLAB)]
            for q in range(2):
                pltpu.make_async_copy(
                    tbl.at[:, pl.ds(0, half_bytes * 128)],
                    stage.at[:, pl.ds(third * _SLAB * 128, half_bytes * 128)],
                    sems[third][q]).wait()
            lanevec = (rv & 127) + (iota * 128) + (third * _SLAB * 128)
            out_sl = pl.ds(t * _SLAB, _SLAB)
            for d in range(16):
                row = plsc.load_gather(
                    stage, [jnp.full((16,), d, jnp.int32), lanevec])
                fm[d, out_sl] = row

        work = ((idx_u, eu_hbm, fm_u), (idx_i, ei_hbm, fm_i),
                (idx_j, ei_hbm, fm_j))

        for idx_v, tbl, fm in work:
            fire(idx_v, tbl, 0, 0)
            fire(idx_v, tbl, 1, 1)

            def body(p, _, idx_v=idx_v, tbl=tbl, fm=fm):
                t0 = p * 3
                for k in range(3):
                    t = t0 + k
                    nxt = t + 2

                    @pl.when(nxt < n_slabs)
                    def _(t=t, nxt=nxt, k=k):
                        fire(idx_v, tbl, nxt, (k + 2) % 3)

                    @pl.when(t < n_slabs)
                    def _(t=t, k=k):
                        drain_extract(idx_v, tbl, fm, t, k)

                return 0

            lax.fori_loop(0, (n_slabs + 2) // 3, body, 0)

        out_sl = pl.ds(base, b_per_w)
        pltpu.sync_copy(fm_u, u_out.at[:, out_sl])
        pltpu.sync_copy(fm_i, i_out.at[:, out_sl])
        pltpu.sync_copy(fm_j, j_out.at[:, out_sl])

    return gather_k


def _loss_body(u_ref, i_ref, j_ref, out_ref, acc):
    b = pl.program_id(0)
    nb = pl.num_programs(0)

    @pl.when(b == 0)
    def _init():
        acc[0] = 0.0
        acc[1] = 0.0

    u = u_ref[...]
    i = i_ref[...]
    j = j_ref[...]

    x_ui = jnp.sum(u * i, axis=0, keepdims=True)
    x_uj = jnp.sum(u * j, axis=0, keepdims=True)
    t = x_ui - x_uj
    x = jnp.clip(t, -80.0, 1e8)
    log_prob = jnp.sum(jax.nn.log_sigmoid(x))
    reg = _LAMBDA_PARAM * (jnp.sum(u * u) + jnp.sum(i * i) + jnp.sum(j * j))

    mask = ((t >= -80.0) & (t <= 1e8)).astype(jnp.float32)
    c = jax.nn.sigmoid(-x) * mask
    gu = -c * (i - j) + (2.0 * _LAMBDA_PARAM) * u
    gi = -c * u + (2.0 * _LAMBDA_PARAM) * i
    gj = c * u + (2.0 * _LAMBDA_PARAM) * j

    def _delta(g):
        n = jnp.maximum(jnp.sqrt(jnp.sum(g * g, axis=0, keepdims=True)), 1e-12)
        return (_EPSILON / n) * g

    ua = u + _delta(gu)
    x_ui_a = jnp.sum(ua * (i + _delta(gi)), axis=0, keepdims=True)
    x_uj_a = jnp.sum(ua * (j + _delta(gj)), axis=0, keepdims=True)
    x_a = jnp.clip(x_ui_a - x_uj_a, -80.0, 1e8)
    log_prob_adv = jnp.sum(jax.nn.log_sigmoid(x_a))

    acc[0] += -log_prob + reg
    acc[1] += -_LAMBDA_ADV * log_prob_adv

    @pl.when(b == nb - 1)
    def _emit():
        loss = acc[0]
        out_ref[0] = loss
        out_ref[1] = loss + acc[1]


def _make_tc_loss(B, D, block):
    grid = B // block
    spec = pl.BlockSpec((D, block), lambda b: (0, b))
    return pl.pallas_call(
        _loss_body,
        grid=(grid,),
        in_specs=[spec, spec, spec],
        out_specs=pl.BlockSpec(memory_space=pltpu.SMEM),
        out_shape=jax.ShapeDtypeStruct((2,), jnp.float32),
        scratch_shapes=[pltpu.SMEM((2,), jnp.float32)],
    )


def kernel(user, item_i, item_j, epoch, embed_user, embed_item):
    B, = user.shape
    D = embed_user.shape[1]
    u_fm, i_fm, j_fm = _make_sc_gather(B, D)(
        user.astype(jnp.int32), item_i.astype(jnp.int32),
        item_j.astype(jnp.int32), embed_user.T, embed_item.T)
    out = _make_tc_loss(B, D, B)(u_fm, i_fm, j_fm)
    return jnp.where(epoch < _EPOCH_ADV, out[0], out[1])


# speedup vs baseline: 1.0560x; 1.0243x over previous
import functools

import jax
import jax.numpy as jnp
from jax import lax
from jax.experimental import pallas as pl
from jax.experimental.pallas import tpu as pltpu
from jax.experimental.pallas import tpu_sc as plsc

_LAMBDA_PARAM = 0.001
_LAMBDA_ADV = 1.0
_EPSILON = 0.5
_EPOCH_ADV = 1

_NC = 2
_NS = 16
_NW = _NC * _NS
_SLAB = 16


def _make_sc_gather(B, D):
    b_per_w = B // _NW
    out_t = jax.ShapeDtypeStruct((D, B), jnp.float32)
    mesh = plsc.VectorSubcoreMesh(core_axis_name="c", subcore_axis_name="s")

    @functools.partial(
        pl.kernel,
        mesh=mesh,
        out_type=[out_t, out_t, out_t],
        compiler_params=pltpu.CompilerParams(needs_layout_passes=False),
        scratch_types=[
            pltpu.VMEM((b_per_w,), jnp.int32),
            pltpu.VMEM((b_per_w,), jnp.int32),
            pltpu.VMEM((b_per_w,), jnp.int32),
            pltpu.VMEM((16, 3 * _SLAB * 128), jnp.float32),
            pltpu.VMEM((D, b_per_w), jnp.float32),
            pltpu.VMEM((D, b_per_w), jnp.float32),
            pltpu.VMEM((D, b_per_w), jnp.float32),
            pltpu.SemaphoreType.DMA,
            pltpu.SemaphoreType.DMA,
            pltpu.SemaphoreType.DMA,
            pltpu.SemaphoreType.DMA,
            pltpu.SemaphoreType.DMA,
            pltpu.SemaphoreType.DMA,
        ],
    )
    def gather_k(user_hbm, itemi_hbm, itemj_hbm, eu_hbm, ei_hbm,
                 u_out, i_out, j_out,
                 idx_u, idx_i, idx_j, stage, fm_u, fm_i, fm_j,
                 sem0, sem1, sem2, sem3, sem4, sem5):
        wid = lax.axis_index("s") * _NC + lax.axis_index("c")
        base = wid * b_per_w
        in_sl = pl.ds(base, b_per_w)
        pltpu.sync_copy(user_hbm.at[in_sl], idx_u)
        pltpu.sync_copy(itemi_hbm.at[in_sl], idx_i)
        pltpu.sync_copy(itemj_hbm.at[in_sl], idx_j)
        iota = lax.iota(jnp.int32, 16)
        n_slabs = b_per_w // _SLAB
        sems = ((sem0, sem1), (sem2, sem3), (sem4, sem5))
        half_bytes = _SLAB // 2

        def fire(idx_v, tbl, t, third):
            rv = idx_v[pl.ds(t * _SLAB, _SLAB)]
            r128v = (rv >> 7) << 7
            for s in range(_SLAB):
                r128 = pl.multiple_of(r128v[s], 128)
                pltpu.async_copy(
                    tbl.at[:, pl.ds(r128, 128)],
                    stage.at[:, pl.ds((third * _SLAB + s) * 128, 128)],
                    sems[third][s % 2])

        def drain_extract(idx_v, tbl, fm, t, third):
            rv = idx_v[pl.ds(t * _SLAB, _SLAB)]
            for q in range(2):
                pltpu.make_async_copy(
                    tbl.at[:, pl.ds(0, half_bytes * 128)],
                    stage.at[:, pl.ds(third * _SLAB * 128, half_bytes * 128)],
                    sems[third][q]).wait()
            lanevec = (rv & 127) + (iota * 128) + (third * _SLAB * 128)
            out_sl = pl.ds(t * _SLAB, _SLAB)
            for d in range(16):
                row = plsc.load_gather(
                    stage, [jnp.full((16,), d, jnp.int32), lanevec])
                fm[d, out_sl] = row

        work = ((idx_u, eu_hbm, fm_u, u_out), (idx_i, ei_hbm, fm_i, i_out),
                (idx_j, ei_hbm, fm_j, j_out))

        out_sl = pl.ds(base, b_per_w)
        fire(work[0][0], work[0][1], 0, 0)
        fire(work[0][0], work[0][1], 1, 1)
        for w, (idx_v, tbl, fm, out) in enumerate(work):
            def body(p, _, idx_v=idx_v, tbl=tbl, fm=fm, w=w):
                t0 = p * 3
                for k in range(3):
                    fire(idx_v, tbl, t0 + k + 2, (k + 2 + 2 * w) % 3)
                    drain_extract(idx_v, tbl, fm, t0 + k, (k + 2 * w) % 3)
                return 0

            lax.fori_loop(0, (n_slabs - 2) // 3, body, 0)
            if w + 1 < len(work):
                nidx, ntbl = work[w + 1][0], work[w + 1][1]
                fire(nidx, ntbl, 0, (2 * w + 2) % 3)
                fire(nidx, ntbl, 1, (2 * w + 3) % 3)
            drain_extract(idx_v, tbl, fm, n_slabs - 2, (n_slabs - 2 + 2 * w) % 3)
            drain_extract(idx_v, tbl, fm, n_slabs - 1, (n_slabs - 1 + 2 * w) % 3)
            pltpu.sync_copy(fm, out.at[:, out_sl])

    return gather_k


def _loss_body(u_ref, i_ref, j_ref, out_ref, acc):
    b = pl.program_id(0)
    nb = pl.num_programs(0)

    @pl.when(b == 0)
    def _init():
        acc[0] = 0.0
        acc[1] = 0.0

    u = u_ref[...]
    i = i_ref[...]
    j = j_ref[...]

    x_ui = jnp.sum(u * i, axis=0, keepdims=True)
    x_uj = jnp.sum(u * j, axis=0, keepdims=True)
    t = x_ui - x_uj
    x = jnp.clip(t, -80.0, 1e8)
    log_prob = jnp.sum(jax.nn.log_sigmoid(x))
    reg = _LAMBDA_PARAM * (jnp.sum(u * u) + jnp.sum(i * i) + jnp.sum(j * j))

    mask = ((t >= -80.0) & (t <= 1e8)).astype(jnp.float32)
    c = jax.nn.sigmoid(-x) * mask
    gu = -c * (i - j) + (2.0 * _LAMBDA_PARAM) * u
    gi = -c * u + (2.0 * _LAMBDA_PARAM) * i
    gj = c * u + (2.0 * _LAMBDA_PARAM) * j

    def _delta(g):
        n = jnp.maximum(jnp.sqrt(jnp.sum(g * g, axis=0, keepdims=True)), 1e-12)
        return (_EPSILON / n) * g

    ua = u + _delta(gu)
    x_ui_a = jnp.sum(ua * (i + _delta(gi)), axis=0, keepdims=True)
    x_uj_a = jnp.sum(ua * (j + _delta(gj)), axis=0, keepdims=True)
    x_a = jnp.clip(x_ui_a - x_uj_a, -80.0, 1e8)
    log_prob_adv = jnp.sum(jax.nn.log_sigmoid(x_a))

    acc[0] += -log_prob + reg
    acc[1] += -_LAMBDA_ADV * log_prob_adv

    @pl.when(b == nb - 1)
    def _emit():
        loss = acc[0]
        out_ref[0] = loss
        out_ref[1] = loss + acc[1]


def _make_tc_loss(B, D, block):
    grid = B // block
    spec = pl.BlockSpec((D, block), lambda b: (0, b))
    return pl.pallas_call(
        _loss_body,
        grid=(grid,),
        in_specs=[spec, spec, spec],
        out_specs=pl.BlockSpec(memory_space=pltpu.SMEM),
        out_shape=jax.ShapeDtypeStruct((2,), jnp.float32),
        scratch_shapes=[pltpu.SMEM((2,), jnp.float32)],
    )


def kernel(user, item_i, item_j, epoch, embed_user, embed_item):
    B, = user.shape
    D = embed_user.shape[1]
    u_fm, i_fm, j_fm = _make_sc_gather(B, D)(
        user.astype(jnp.int32), item_i.astype(jnp.int32),
        item_j.astype(jnp.int32), embed_user.T, embed_item.T)
    out = _make_tc_loss(B, D, B)(u_fm, i_fm, j_fm)
    return jnp.where(epoch < _EPOCH_ADV, out[0], out[1])
